# baseline (speedup 1.0000x reference)
import functools

import jax
import jax.numpy as jnp
from jax import lax
from jax.experimental import pallas as pl
from jax.experimental.pallas import tpu as pltpu
from jax.experimental.pallas import tpu_sc as plsc

_HEADS = 12


def _reduce_body(a_ref, out_ref):
    m = pl.program_id(0)
    r = pl.program_id(1)

    @pl.when(jnp.logical_and(m % _HEADS == 0, r == 0))
    def _():
        out_ref[...] = jnp.zeros_like(out_ref)

    data = a_ref[0]
    R, S = data.shape
    rows = jax.lax.broadcasted_iota(jnp.int32, (R, S), 0) + r * R
    cols = jax.lax.broadcasted_iota(jnp.int32, (R, S), 1)
    contrib = jnp.where(rows == cols, 0.0, data)
    out_ref[0] += jnp.sum(contrib, axis=0, keepdims=True)


def _rank_body(arow_ref, acol_ref, rank_ref):
    sb = pl.program_id(1)
    a_row = arow_ref[0]
    a_col = acol_ref[...]
    SB = a_col.shape[0]
    S = a_row.shape[1]
    s_glob = jax.lax.broadcasted_iota(jnp.int32, (SB, S), 0) + sb * SB
    j_glob = jax.lax.broadcasted_iota(jnp.int32, (SB, S), 1)
    gt = a_row > a_col
    tie = jnp.logical_and(a_row == a_col, j_glob < s_glob)
    cmp = jnp.where(jnp.logical_or(gt, tie), 1.0, 0.0)
    rank_ref[...] = jnp.sum(cmp, axis=1, keepdims=True).astype(jnp.int32)


def _mul_body(gate_ref, x_ref, out_ref):
    out_ref[0] = x_ref[0] * gate_ref[...]


def _sc_gather(rank_flat, w):
    BS = rank_flat.shape[0]
    S = w.shape[0]
    info = plsc.get_sparse_core_info()
    NC, NS, L = info.num_cores, info.num_subcores, info.num_lanes
    NW = NC * NS
    per_w = BS // NW

    mesh = plsc.VectorSubcoreMesh(core_axis_name="c", subcore_axis_name="s")

    @functools.partial(
        pl.kernel,
        out_type=jax.ShapeDtypeStruct((BS,), jnp.float32),
        mesh=mesh,
        scratch_types=[
            pltpu.VMEM((S,), jnp.float32),
            pltpu.VMEM((per_w,), jnp.int32),
            pltpu.VMEM((per_w,), jnp.float32),
        ],
        compiler_params=pltpu.CompilerParams(needs_layout_passes=False),
    )
    def k(rank_hbm, w_hbm, gate_hbm, w_v, r_v, g_v):
        wid = lax.axis_index("s") * NC + lax.axis_index("c")
        base = wid * per_w
        pltpu.sync_copy(w_hbm, w_v)
        pltpu.sync_copy(rank_hbm.at[pl.ds(base, per_w)], r_v)
        for v in range(per_w // L):
            idx = r_v[pl.ds(v * L, L)]
            g_v[pl.ds(v * L, L)] = plsc.load_gather(w_v, [idx])
        pltpu.sync_copy(g_v, gate_hbm.at[pl.ds(base, per_w)])

    return k(rank_flat, w)


def kernel(x, atten, W):
    B, S, D = x.shape
    BH = atten.shape[0]
    R = 2048
    SB = 256
    nr = S // R
    nsb = S // SB

    attended = pl.pallas_call(
        _reduce_body,
        grid=(BH, nr),
        in_specs=[pl.BlockSpec((1, R, S), lambda m, r: (m, r, 0))],
        out_specs=pl.BlockSpec((1, 1, S), lambda m, r: (m // _HEADS, 0, 0)),
        out_shape=jax.ShapeDtypeStruct((B, 1, S), jnp.float32),
    )(atten)

    a_col = attended.reshape(B * S, 1)

    rank = pl.pallas_call(
        _rank_body,
        grid=(B, nsb),
        in_specs=[
            pl.BlockSpec((1, 1, S), lambda b, s: (b, 0, 0)),
            pl.BlockSpec((SB, 1), lambda b, s, _n=nsb: (b * _n + s, 0)),
        ],
        out_specs=pl.BlockSpec((SB, 1), lambda b, s, _n=nsb: (b * _n + s, 0)),
        out_shape=jax.ShapeDtypeStruct((B * S, 1), jnp.int32),
    )(attended, a_col)

    gate = _sc_gather(rank.reshape(B * S), W)
    gate_col = gate.reshape(B * S, 1)

    out = pl.pallas_call(
        _mul_body,
        grid=(B, nsb),
        in_specs=[
            pl.BlockSpec((SB, 1), lambda b, s, _n=nsb: (b * _n + s, 0)),
            pl.BlockSpec((1, SB, D), lambda b, s: (b, s, 0)),
        ],
        out_specs=pl.BlockSpec((1, SB, D), lambda b, s: (b, s, 0)),
        out_shape=jax.ShapeDtypeStruct((B, S, D), jnp.float32),
    )(gate_col, x)
    return out
